# baseline (speedup 1.0000x reference)
import functools

import jax
import jax.numpy as jnp
from jax import lax
from jax.experimental import pallas as pl
from jax.experimental.pallas import tpu as pltpu
from jax.experimental.pallas import tpu_sc as plsc

_INFO = plsc.get_sparse_core_info()
_NC, _NS = _INFO.num_cores, _INFO.num_subcores
_NW = _NC * _NS

_CHUNK = 100
_EMBED = 32


def _make_kernel(n_rows, maxlen, embed):
    chunks_total = n_rows
    per_w = chunks_total // _NW
    pos_splits = maxlen // _CHUNK

    mesh = plsc.VectorSubcoreMesh(core_axis_name="c", subcore_axis_name="s")

    @functools.partial(
        pl.kernel,
        out_type=jax.ShapeDtypeStruct((n_rows, _CHUNK, embed), jnp.float32),
        mesh=mesh,
        scratch_types=[
            pltpu.VMEM((_CHUNK,), jnp.int32),
            pltpu.VMEM((pos_splits, _CHUNK), jnp.int32),
            pltpu.VMEM((_CHUNK, embed), jnp.float32),
            pltpu.SemaphoreType.DMA,
        ],
        compiler_params=pltpu.CompilerParams(use_tc_tiling_on_sc=False),
    )
    def k(x_hbm, tok_hbm, pos_hbm, pidx_hbm, out_hbm, idx_v, pidx_v, rows_v, sem):
        wid = lax.axis_index("s") * _NC + lax.axis_index("c")
        base = wid * per_w
        pltpu.sync_copy(pidx_hbm, pidx_v)

        def step(c, carry):
            row = base + c
            parity = lax.rem(row, pos_splits)
            pltpu.sync_copy(x_hbm.at[row], idx_v)
            pltpu.async_copy(tok_hbm.at[idx_v], rows_v, sem).wait()
            pltpu.async_copy(
                pos_hbm.at[pidx_v.at[parity]], rows_v, sem, add=True
            ).wait()
            pltpu.sync_copy(rows_v, out_hbm.at[row])
            return carry

        lax.fori_loop(0, per_w, step, 0)

    return k


def kernel(x, token_table, pos_table):
    batch, maxlen = x.shape
    embed = token_table.shape[-1]
    n_rows = batch * maxlen // _CHUNK
    x2 = x.astype(jnp.int32).reshape(n_rows, _CHUNK)
    pos_idx = jnp.arange(maxlen, dtype=jnp.int32).reshape(-1, _CHUNK)
    k = _make_kernel(n_rows, maxlen, embed)
    out = k(x2, token_table, pos_table, pos_idx)
    return out.reshape(batch, maxlen, embed)


# speedup vs baseline: 1.1245x; 1.1245x over previous
import functools

import jax
import jax.numpy as jnp
from jax import lax
from jax.experimental import pallas as pl
from jax.experimental.pallas import tpu as pltpu
from jax.experimental.pallas import tpu_sc as plsc

_INFO = plsc.get_sparse_core_info()
_NC, _NS = _INFO.num_cores, _INFO.num_subcores
_NW = _NC * _NS

_CHUNK = 100
_NBUF = 8


def _make_kernel(n_rows, maxlen, embed):
    per_w = n_rows // _NW
    pos_splits = maxlen // _CHUNK
    assert per_w % _NBUF == 0

    mesh = plsc.VectorSubcoreMesh(core_axis_name="c", subcore_axis_name="s")

    @functools.partial(
        pl.kernel,
        out_type=jax.ShapeDtypeStruct((n_rows, _CHUNK, embed), jnp.float32),
        mesh=mesh,
        scratch_types=[
            pltpu.VMEM((per_w, _CHUNK), jnp.int32),
            pltpu.VMEM((pos_splits, _CHUNK), jnp.int32),
            pltpu.VMEM((_NBUF, _CHUNK, embed), jnp.float32),
        ]
        + [pltpu.SemaphoreType.DMA] * _NBUF,
        compiler_params=pltpu.CompilerParams(use_tc_tiling_on_sc=False),
    )
    def k(x_hbm, tok_hbm, pos_hbm, pidx_hbm, out_hbm, idx_v, pidx_v, rows_v,
          *sems):
        wid = lax.axis_index("s") * _NC + lax.axis_index("c")
        base = wid * per_w
        pltpu.sync_copy(x_hbm.at[wid], idx_v)
        pltpu.sync_copy(pidx_hbm, pidx_v)

        @pl.loop(0, per_w, step=_NBUF)
        def group(g):
            toks = []
            for b in range(_NBUF):
                toks.append(pltpu.async_copy(
                    tok_hbm.at[idx_v.at[g + b]], rows_v.at[b], sems[b]))
            poss = []
            for b in range(_NBUF):
                toks[b].wait()
                parity = lax.rem(g + b, pos_splits)
                poss.append(pltpu.async_copy(
                    pos_hbm.at[pidx_v.at[parity]], rows_v.at[b], sems[b],
                    add=True))
            outs = []
            for b in range(_NBUF):
                poss[b].wait()
                outs.append(pltpu.async_copy(
                    rows_v.at[b], out_hbm.at[base + g + b], sems[b]))
            for b in range(_NBUF):
                outs[b].wait()

    return k


def kernel(x, token_table, pos_table):
    batch, maxlen = x.shape
    embed = token_table.shape[-1]
    n_rows = batch * maxlen // _CHUNK
    x3 = x.astype(jnp.int32).reshape(_NW, n_rows // _NW, _CHUNK)
    pos_idx = jnp.arange(maxlen, dtype=jnp.int32).reshape(-1, _CHUNK)
    k = _make_kernel(n_rows, maxlen, embed)
    out = k(x3, token_table, pos_table, pos_idx)
    return out.reshape(batch, maxlen, embed)
